# baseline (speedup 1.0000x reference)
import jax
import jax.numpy as jnp
from jax.experimental import pallas as pl

CONF_THRES = 0.25
IOU_THRES = 0.45
MAX_NMS = 1000
MAX_DET = 300
MAX_WH = 4096.0

_N = 1024
_R = 512
_ROWS = 4000
_RPAD = 4096
_NR = 5
_MPAD = 5120


def _scores_body(pred_ref, scores_ref, cls_ref):
    for r in range(_NR):
        blk = pred_ref[0, pl.ds(r * _ROWS, _ROWS), :]
        obj = blk[:, 4:5]
        prod = blk * obj
        lane = jax.lax.broadcasted_iota(jnp.int32, (_ROWS, 85), 1)
        masked = jnp.where(lane >= 5, prod, -jnp.inf)
        conf = jnp.max(masked, axis=1, keepdims=True)
        cand = jnp.where(masked >= conf,
                         lane.astype(jnp.float32), 1e9)
        cls_id = jnp.min(cand, axis=1) - 5.0
        confv = conf[:, 0]
        scores_ref[0, r, :_ROWS] = jnp.where(confv > CONF_THRES, confv, -1.0)
        scores_ref[0, r, _ROWS:] = jnp.full((_RPAD - _ROWS,), -2.0)
        cls_ref[0, r, :_ROWS] = cls_id
        cls_ref[0, r, _ROWS:] = jnp.zeros((_RPAD - _ROWS,))


def _compute_scores(pred):
    B, N, C = pred.shape
    spec = pl.BlockSpec((1, _NR, _RPAD), lambda b: (b, 0, 0))
    shp = jax.ShapeDtypeStruct((B, _NR, _RPAD), jnp.float32)
    return pl.pallas_call(
        _scores_body,
        grid=(B,),
        in_specs=[pl.BlockSpec((1, N, C), lambda b: (b, 0, 0))],
        out_specs=[spec] * 2,
        out_shape=[shp] * 2,
    )(pred)


def _nms_body(d_ref, out_ref):
    d = d_ref[0]

    sub = jax.lax.broadcasted_iota(jnp.int32, (_N, _N), 0)
    lan = jax.lax.broadcasted_iota(jnp.int32, (_N, _N), 1)
    eq = sub == lan

    def to_col(r):
        return jnp.sum(jnp.where(eq, r, 0.0), axis=1, keepdims=True)

    xr, yr, wr, hr = d[0:1, :], d[1:2, :], d[2:3, :], d[3:4, :]
    scr, clr = d[4:5, :], d[5:6, :]
    offr = clr * MAX_WH
    rx1 = (xr - wr / 2.0) + offr
    ry1 = (yr - hr / 2.0) + offr
    rx2 = (xr + wr / 2.0) + offr
    ry2 = (yr + hr / 2.0) + offr
    area_r = (rx2 - rx1) * (ry2 - ry1)

    xc, yc, wc, hc = to_col(xr), to_col(yr), to_col(wr), to_col(hr)
    scc, clc = to_col(scr), to_col(clr)
    offc = clc * MAX_WH
    cx1 = (xc - wc / 2.0) + offc
    cy1 = (yc - hc / 2.0) + offc
    cx2 = (xc + wc / 2.0) + offc
    cy2 = (yc + hc / 2.0) + offc
    area_c = (cx2 - cx1) * (cy2 - cy1)

    iw = jnp.clip(jnp.minimum(cx2, rx2) - jnp.maximum(cx1, rx1), 0.0, None)
    ih = jnp.clip(jnp.minimum(cy2, ry2) - jnp.maximum(cy1, ry1), 0.0, None)
    inter = iw * ih
    iou = inter / (area_c + area_r - inter + 1e-9)

    lower = sub < lan
    Mf = jnp.where(lower & (iou > IOU_THRES), 1.0, 0.0)

    alive = jnp.where(scr > CONF_THRES, 1.0, 0.0)

    def cond(carry):
        _, changed = carry
        return changed

    def body(carry):
        S, _ = carry
        supp = jnp.dot(S, Mf, preferred_element_type=jnp.float32)
        S_new = alive * jnp.where(supp < 0.5, 1.0, 0.0)
        return S_new, jnp.any(S_new != S)

    keep, _ = jax.lax.while_loop(cond, body, (alive, jnp.bool_(True)))

    LTf = jnp.where(lower, 1.0, 0.0)
    rank = jnp.dot(keep, LTf, preferred_element_type=jnp.float32)

    rsub = jax.lax.broadcasted_iota(jnp.int32, (_R, _N), 0)
    PT = jnp.where((rank.astype(jnp.int32) == rsub) & (keep > 0.5), 1.0, 0.0)

    zc = jnp.zeros_like(xc)
    Dcol = jnp.concatenate(
        [xc - wc / 2.0, yc - hc / 2.0, xc + wc / 2.0, yc + hc / 2.0,
         scc, clc, zc, zc], axis=1)

    out_ref[0] = jnp.dot(PT, Dcol, preferred_element_type=jnp.float32)


def _run_nms(d):
    B = d.shape[0]
    return pl.pallas_call(
        _nms_body,
        grid=(B,),
        in_specs=[pl.BlockSpec((1, 6, _N), lambda b: (b, 0, 0))],
        out_specs=pl.BlockSpec((1, _R, 8), lambda b: (b, 0, 0)),
        out_shape=jax.ShapeDtypeStruct((B, _R, 8), jnp.float32),
    )(d)


def kernel(x):
    pred = x[0]
    B, N, _ = pred.shape
    scores, cls_id = _compute_scores(pred)

    v1, li = jax.lax.top_k(scores, MAX_NMS)
    c1 = jnp.take_along_axis(cls_id, li, axis=2)
    g1 = li + (jnp.arange(_NR, dtype=li.dtype) * _ROWS)[None, :, None]
    mpad = _MPAD - _NR * MAX_NMS
    vm = jnp.pad(v1.reshape(B, _NR * MAX_NMS), ((0, 0), (0, mpad)),
                 constant_values=-2.0)
    cm = jnp.pad(c1.reshape(B, _NR * MAX_NMS), ((0, 0), (0, mpad)))
    gm = jnp.pad(g1.reshape(B, _NR * MAX_NMS), ((0, 0), (0, mpad)))

    sc, i2 = jax.lax.top_k(vm, MAX_NMS)
    idx = jnp.take_along_axis(gm, i2, axis=1)
    csel = jnp.take_along_axis(cm, i2, axis=1)
    rows = jnp.take_along_axis(pred, idx[..., None], axis=1)
    xs, ys, ws, hs = (rows[..., 0], rows[..., 1],
                      rows[..., 2], rows[..., 3])

    pad = _N - MAX_NMS
    sc = jnp.pad(sc, ((0, 0), (0, pad)), constant_values=-1.0)
    csel, xs, ys, ws, hs = [jnp.pad(a, ((0, 0), (0, pad)))
                            for a in (csel, xs, ys, ws, hs)]
    d = jnp.stack([xs, ys, ws, hs, sc, csel], axis=1)

    out = _run_nms(d)
    return out[:, :MAX_DET, :6]


# speedup vs baseline: 1.6813x; 1.6813x over previous
import jax
import jax.numpy as jnp
from jax.experimental import pallas as pl

CONF_THRES = 0.25
IOU_THRES = 0.45
MAX_NMS = 1000
MAX_DET = 300
MAX_WH = 4096.0

_N = 1024
_R = 512
_ROWS = 4000
_RPAD = 4096
_NR = 5
_MPAD = 5120


def _filter_body(conf_ref, cls_ref, bx_ref, by_ref, bw_ref, bh_ref,
                 scores_ref, ocls_ref, ox_ref, oy_ref, ow_ref, oh_ref):
    pad = jnp.full((_RPAD - _ROWS,), -2.0)
    zpad = jnp.zeros((_RPAD - _ROWS,))
    for b in range(conf_ref.shape[0]):
        for r in range(_NR):
            sl = pl.ds(r * _ROWS, _ROWS)
            confv = conf_ref[b, sl]
            scores_ref[b, r, :_ROWS] = jnp.where(confv > CONF_THRES,
                                                 confv, -1.0)
            scores_ref[b, r, _ROWS:] = pad
            ocls_ref[b, r, :_ROWS] = cls_ref[b, sl]
            ocls_ref[b, r, _ROWS:] = zpad
            ox_ref[b, r, :_ROWS] = bx_ref[b, sl]
            ox_ref[b, r, _ROWS:] = zpad
            oy_ref[b, r, :_ROWS] = by_ref[b, sl]
            oy_ref[b, r, _ROWS:] = zpad
            ow_ref[b, r, :_ROWS] = bw_ref[b, sl]
            ow_ref[b, r, _ROWS:] = zpad
            oh_ref[b, r, :_ROWS] = bh_ref[b, sl]
            oh_ref[b, r, _ROWS:] = zpad


def _filter_chunk(conf, cls_id, bx, by, bw, bh):
    B, N = conf.shape
    ispec = pl.BlockSpec((B, N), lambda: (0, 0))
    ospec = pl.BlockSpec((B, _NR, _RPAD), lambda: (0, 0, 0))
    shp = jax.ShapeDtypeStruct((B, _NR, _RPAD), jnp.float32)
    return pl.pallas_call(
        _filter_body,
        grid=(),
        in_specs=[ispec] * 6,
        out_specs=[ospec] * 6,
        out_shape=[shp] * 6,
    )(conf, cls_id, bx, by, bw, bh)


def _nms_body(d_ref, out_ref):
    d = d_ref[0]

    sub = jax.lax.broadcasted_iota(jnp.int32, (_N, _N), 0)
    lan = jax.lax.broadcasted_iota(jnp.int32, (_N, _N), 1)
    eq = sub == lan

    def to_col(r):
        return jnp.sum(jnp.where(eq, r, 0.0), axis=1, keepdims=True)

    xr, yr, wr, hr = d[0:1, :], d[1:2, :], d[2:3, :], d[3:4, :]
    scr, clr = d[4:5, :], d[5:6, :]
    offr = clr * MAX_WH
    rx1 = (xr - wr / 2.0) + offr
    ry1 = (yr - hr / 2.0) + offr
    rx2 = (xr + wr / 2.0) + offr
    ry2 = (yr + hr / 2.0) + offr
    area_r = (rx2 - rx1) * (ry2 - ry1)

    xc, yc, wc, hc = to_col(xr), to_col(yr), to_col(wr), to_col(hr)
    scc, clc = to_col(scr), to_col(clr)
    offc = clc * MAX_WH
    cx1 = (xc - wc / 2.0) + offc
    cy1 = (yc - hc / 2.0) + offc
    cx2 = (xc + wc / 2.0) + offc
    cy2 = (yc + hc / 2.0) + offc
    area_c = (cx2 - cx1) * (cy2 - cy1)

    iw = jnp.clip(jnp.minimum(cx2, rx2) - jnp.maximum(cx1, rx1), 0.0, None)
    ih = jnp.clip(jnp.minimum(cy2, ry2) - jnp.maximum(cy1, ry1), 0.0, None)
    inter = iw * ih
    iou = inter / (area_c + area_r - inter + 1e-9)

    lower = sub < lan
    Mf = jnp.where(lower & (iou > IOU_THRES), 1.0, 0.0)

    alive = jnp.where(scr > CONF_THRES, 1.0, 0.0)

    def cond(carry):
        _, changed = carry
        return changed

    def body(carry):
        S, _ = carry
        supp = jnp.dot(S, Mf, preferred_element_type=jnp.float32)
        S_new = alive * jnp.where(supp < 0.5, 1.0, 0.0)
        return S_new, jnp.any(S_new != S)

    keep, _ = jax.lax.while_loop(cond, body, (alive, jnp.bool_(True)))

    LTf = jnp.where(lower, 1.0, 0.0)
    rank = jnp.dot(keep, LTf, preferred_element_type=jnp.float32)

    rsub = jax.lax.broadcasted_iota(jnp.int32, (_R, _N), 0)
    PT = jnp.where((rank.astype(jnp.int32) == rsub) & (keep > 0.5), 1.0, 0.0)

    zc = jnp.zeros_like(xc)
    Dcol = jnp.concatenate(
        [xc - wc / 2.0, yc - hc / 2.0, xc + wc / 2.0, yc + hc / 2.0,
         scc, clc, zc, zc], axis=1)

    out_ref[0] = jnp.dot(PT, Dcol, preferred_element_type=jnp.float32)


def _run_nms(d):
    B = d.shape[0]
    return pl.pallas_call(
        _nms_body,
        grid=(B,),
        in_specs=[pl.BlockSpec((1, 6, _N), lambda b: (b, 0, 0))],
        out_specs=pl.BlockSpec((1, _R, 8), lambda b: (b, 0, 0)),
        out_shape=jax.ShapeDtypeStruct((B, _R, 8), jnp.float32),
    )(d)


def kernel(x):
    pred = x[0]
    B, N, _ = pred.shape

    obj = pred[..., 4]
    cls_conf = pred[..., 5:] * obj[..., None]
    conf = jnp.max(cls_conf, axis=-1)
    cls_id = jnp.argmax(cls_conf, axis=-1).astype(jnp.float32)
    bx, by, bw, bh = (pred[..., 0], pred[..., 1],
                      pred[..., 2], pred[..., 3])

    scores, cm0, xm0, ym0, wm0, hm0 = _filter_chunk(
        conf, cls_id, bx, by, bw, bh)

    v1, li = jax.lax.top_k(scores, MAX_NMS)
    fields1 = [jnp.take_along_axis(a, li, axis=2).reshape(B, _NR * MAX_NMS)
               for a in (cm0, xm0, ym0, wm0, hm0)]
    mpad = _MPAD - _NR * MAX_NMS
    vm = jnp.pad(v1.reshape(B, _NR * MAX_NMS), ((0, 0), (0, mpad)),
                 constant_values=-2.0)
    fields1 = [jnp.pad(a, ((0, 0), (0, mpad))) for a in fields1]

    sc, i2 = jax.lax.top_k(vm, MAX_NMS)
    csel, xs, ys, ws, hs = [jnp.take_along_axis(a, i2, axis=1)
                            for a in fields1]

    pad = _N - MAX_NMS
    sc = jnp.pad(sc, ((0, 0), (0, pad)), constant_values=-1.0)
    csel, xs, ys, ws, hs = [jnp.pad(a, ((0, 0), (0, pad)))
                            for a in (csel, xs, ys, ws, hs)]
    d = jnp.stack([xs, ys, ws, hs, sc, csel], axis=1)

    out = _run_nms(d)
    return out[:, :MAX_DET, :6]


# speedup vs baseline: 1.9318x; 1.1490x over previous
import jax
import jax.numpy as jnp
from jax.experimental import pallas as pl

CONF_THRES = 0.25
IOU_THRES = 0.45
MAX_NMS = 1000
MAX_DET = 300
MAX_WH = 4096.0

_N = 1024
_R = 512
_ROWS = 4000
_RPAD = 4096
_NR = 5
_MPAD = 5120


def _filter_body(conf_ref, cls_ref, bx_ref, by_ref, bw_ref, bh_ref,
                 scores_ref, ocls_ref, ox_ref, oy_ref, ow_ref, oh_ref):
    pad = jnp.full((_RPAD - _ROWS,), -2.0)
    zpad = jnp.zeros((_RPAD - _ROWS,))
    for b in range(conf_ref.shape[0]):
        for r in range(_NR):
            sl = pl.ds(r * _ROWS, _ROWS)
            confv = conf_ref[b, sl]
            scores_ref[b, r, :_ROWS] = jnp.where(confv > CONF_THRES,
                                                 confv, -1.0)
            scores_ref[b, r, _ROWS:] = pad
            ocls_ref[b, r, :_ROWS] = cls_ref[b, sl]
            ocls_ref[b, r, _ROWS:] = zpad
            ox_ref[b, r, :_ROWS] = bx_ref[b, sl]
            ox_ref[b, r, _ROWS:] = zpad
            oy_ref[b, r, :_ROWS] = by_ref[b, sl]
            oy_ref[b, r, _ROWS:] = zpad
            ow_ref[b, r, :_ROWS] = bw_ref[b, sl]
            ow_ref[b, r, _ROWS:] = zpad
            oh_ref[b, r, :_ROWS] = bh_ref[b, sl]
            oh_ref[b, r, _ROWS:] = zpad


def _filter_chunk(conf, cls_id, bx, by, bw, bh):
    B, N = conf.shape
    ispec = pl.BlockSpec((B, N), lambda: (0, 0))
    ospec = pl.BlockSpec((B, _NR, _RPAD), lambda: (0, 0, 0))
    shp = jax.ShapeDtypeStruct((B, _NR, _RPAD), jnp.float32)
    return pl.pallas_call(
        _filter_body,
        grid=(),
        in_specs=[ispec] * 6,
        out_specs=[ospec] * 6,
        out_shape=[shp] * 6,
    )(conf, cls_id, bx, by, bw, bh)


def _nms_body(d_ref, out_ref):
    d = d_ref[0]

    sub = jax.lax.broadcasted_iota(jnp.int32, (_N, _N), 0)
    lan = jax.lax.broadcasted_iota(jnp.int32, (_N, _N), 1)
    eq = sub == lan

    def to_col(r):
        return jnp.sum(jnp.where(eq, r, 0.0), axis=1, keepdims=True)

    xr, yr, wr, hr = d[0:1, :], d[1:2, :], d[2:3, :], d[3:4, :]
    scr, clr = d[4:5, :], d[5:6, :]
    offr = clr * MAX_WH
    rx1 = (xr - wr / 2.0) + offr
    ry1 = (yr - hr / 2.0) + offr
    rx2 = (xr + wr / 2.0) + offr
    ry2 = (yr + hr / 2.0) + offr
    area_r = (rx2 - rx1) * (ry2 - ry1)

    xc, yc, wc, hc = to_col(xr), to_col(yr), to_col(wr), to_col(hr)
    scc, clc = to_col(scr), to_col(clr)
    offc = clc * MAX_WH
    cx1 = (xc - wc / 2.0) + offc
    cy1 = (yc - hc / 2.0) + offc
    cx2 = (xc + wc / 2.0) + offc
    cy2 = (yc + hc / 2.0) + offc
    area_c = (cx2 - cx1) * (cy2 - cy1)

    iw = jnp.clip(jnp.minimum(cx2, rx2) - jnp.maximum(cx1, rx1), 0.0, None)
    ih = jnp.clip(jnp.minimum(cy2, ry2) - jnp.maximum(cy1, ry1), 0.0, None)
    inter = iw * ih
    iou = inter / (area_c + area_r - inter + 1e-9)

    lower = sub < lan
    Mf = jnp.where(lower & (iou > IOU_THRES), 1.0, 0.0)

    alive = jnp.where(scr > CONF_THRES, 1.0, 0.0)

    def cond(carry):
        _, changed = carry
        return changed

    def body(carry):
        S, _ = carry
        supp = jnp.dot(S, Mf, preferred_element_type=jnp.float32)
        S_new = alive * jnp.where(supp < 0.5, 1.0, 0.0)
        return S_new, jnp.any(S_new != S)

    keep, _ = jax.lax.while_loop(cond, body, (alive, jnp.bool_(True)))

    LTf = jnp.where(lower, 1.0, 0.0)
    rank = jnp.dot(keep, LTf, preferred_element_type=jnp.float32)

    rsub = jax.lax.broadcasted_iota(jnp.int32, (_R, _N), 0)
    PT = jnp.where((rank.astype(jnp.int32) == rsub) & (keep > 0.5), 1.0, 0.0)

    zc = jnp.zeros_like(xc)
    Dcol = jnp.concatenate(
        [xc - wc / 2.0, yc - hc / 2.0, xc + wc / 2.0, yc + hc / 2.0,
         scc, clc, zc, zc], axis=1)

    out_ref[0] = jnp.dot(PT, Dcol, preferred_element_type=jnp.float32)


def _run_nms(d):
    B = d.shape[0]
    return pl.pallas_call(
        _nms_body,
        grid=(B,),
        in_specs=[pl.BlockSpec((1, 6, _N), lambda b: (b, 0, 0))],
        out_specs=pl.BlockSpec((1, _R, 8), lambda b: (b, 0, 0)),
        out_shape=jax.ShapeDtypeStruct((B, _R, 8), jnp.float32),
    )(d)


def kernel(x):
    pred = x[0]
    B, N, _ = pred.shape

    obj = pred[..., 4]
    cls_conf = pred[..., 5:] * obj[..., None]
    conf = jnp.max(cls_conf, axis=-1)
    cls_id = jnp.argmax(cls_conf, axis=-1).astype(jnp.float32)
    bx, by, bw, bh = (pred[..., 0], pred[..., 1],
                      pred[..., 2], pred[..., 3])

    scores, cm0, xm0, ym0, wm0, hm0 = _filter_chunk(
        conf, cls_id, bx, by, bw, bh)

    v1, li = jax.lax.top_k(scores, MAX_NMS)
    g1 = li + (jnp.arange(_NR, dtype=li.dtype) * _RPAD)[None, :, None]
    mpad = _MPAD - _NR * MAX_NMS
    vm = jnp.pad(v1.reshape(B, _NR * MAX_NMS), ((0, 0), (0, mpad)),
                 constant_values=-2.0)
    gm = jnp.pad(g1.reshape(B, _NR * MAX_NMS), ((0, 0), (0, mpad)))

    sc, i2 = jax.lax.top_k(vm, MAX_NMS)
    idx = jnp.take_along_axis(gm, i2, axis=1)
    csel, xs, ys, ws, hs = [
        jnp.take_along_axis(a.reshape(B, _NR * _RPAD), idx, axis=1)
        for a in (cm0, xm0, ym0, wm0, hm0)]

    pad = _N - MAX_NMS
    sc = jnp.pad(sc, ((0, 0), (0, pad)), constant_values=-1.0)
    csel, xs, ys, ws, hs = [jnp.pad(a, ((0, 0), (0, pad)))
                            for a in (csel, xs, ys, ws, hs)]
    d = jnp.stack([xs, ys, ws, hs, sc, csel], axis=1)

    out = _run_nms(d)
    return out[:, :MAX_DET, :6]
